# baseline (speedup 1.0000x reference)
import jax
import jax.numpy as jnp
from jax import lax
from jax.experimental import pallas as pl
from jax.experimental.pallas import tpu as pltpu
from jax.experimental.pallas import tpu_sc as plsc

N_ROWS = 1_000_000
DIM = 64
BATCH = 819_200
NPAIR = N_ROWS // 2

NUM_CORES = 2
NUM_SUBCORES = 16
LANES = 16
NUM_WORKERS = NUM_CORES * NUM_SUBCORES

C1 = 256
C1_FULL = 999_936 // C1
C1_PER_W = -(-C1_FULL // NUM_WORKERS)
TAIL0 = C1_FULL * C1

B_PER_W = BATCH // NUM_WORKERS
C2 = 256
SUB = 128
KSUB = C2 // SUB
C2_CHUNKS = B_PER_W // C2


def _compact_body(tab_t_hbm, pairs_hbm, in_v, out_v):
    wid = lax.axis_index("s") * NUM_CORES + lax.axis_index("c")
    lane = lax.iota(jnp.int32, LANES)

    def chunk(i, carry):
        cid = wid + i * NUM_WORKERS

        @pl.when(cid < C1_FULL)
        def _():
            i0 = pl.multiple_of(cid * C1, C1)
            pltpu.sync_copy(tab_t_hbm.at[:, pl.ds(i0, C1)], in_v)

            def pair(p, c):
                for half in range(2):
                    tex = 2 * p + half
                    for cc in range(DIM // LANES):
                        src = plsc.load_gather(
                            in_v, [cc * LANES + lane,
                                   jnp.full((LANES,), tex, jnp.int32)])
                        out_v[p, pl.ds(half * DIM + cc * LANES, LANES)] = src
                return c
            lax.fori_loop(0, C1 // 2, pair, 0)
            p0 = pl.multiple_of(i0 // 2, C1 // 2)
            pltpu.sync_copy(out_v, pairs_hbm.at[pl.ds(p0, C1 // 2), :])
        return carry

    lax.fori_loop(0, C1_PER_W, chunk, 0)


def _tail_body(tab_ref, pairs_any, out_ref):
    del pairs_any
    x = tab_ref[...]
    y = x.reshape(DIM, DIM, 2).transpose(1, 2, 0).reshape(DIM, 2 * DIM)
    out_ref[...] = y


def _sample_body(pairs_hbm, param_hbm, out_t_hbm,
                 param_v, w_v, s0_v, s1_v, idxa_v, idxb_v,
                 bufa_v, bufb_v, res_v, sem):
    wid = lax.axis_index("s") * NUM_CORES + lax.axis_index("c")
    base = wid * B_PER_W
    scale = jnp.float32(N_ROWS - 1)
    lane = lax.iota(jnp.int32, LANES)

    def chunk(g, carry):
        off = pl.multiple_of(base + g * C2, C2)
        pltpu.sync_copy(param_hbm.at[pl.ds(off, C2)], param_v)

        for j in range(C2 // LANES):
            p = param_v[pl.ds(j * LANES, LANES)]
            t = jnp.minimum(jnp.maximum(p, 0.0), 1.0) * scale
            i0 = t.astype(jnp.int32)
            i1 = jnp.minimum(i0 + 1, N_ROWS - 1)
            w = t - i0.astype(jnp.float32)
            k, r = divmod(j * LANES, SUB)
            idxa_v[k, pl.ds(r, LANES)] = lax.shift_right_logical(i0, 1)
            idxb_v[k, pl.ds(r, LANES)] = lax.shift_right_logical(i1, 1)
            s0_v[pl.ds(j * LANES, LANES)] = lax.shift_left(
                jnp.bitwise_and(i0, 1), 6)
            s1_v[pl.ds(j * LANES, LANES)] = lax.shift_left(
                jnp.bitwise_and(i1, 1), 6)
            w_v[pl.ds(j * LANES, LANES)] = w

        copies = []
        for k in range(KSUB):
            copies.append(
                pltpu.async_copy(pairs_hbm.at[idxa_v.at[k]], bufa_v.at[k], sem))
            copies.append(
                pltpu.async_copy(pairs_hbm.at[idxb_v.at[k]], bufb_v.at[k], sem))
        for cp in copies:
            cp.wait()

        for k in range(KSUB):
            def row16(r16, c, _k=k):
                rr = _k * SUB + r16 * LANES
                w16 = w_v[pl.ds(rr, LANES)]
                s0_16 = s0_v[pl.ds(rr, LANES)]
                s1_16 = s1_v[pl.ds(rr, LANES)]
                for j in range(LANES):
                    r = r16 * LANES + j
                    q = rr + j
                    wb = w16.at[jnp.full((LANES,), j, jnp.int32)].get(
                        mode="promise_in_bounds")
                    one_m = 1.0 - wb
                    s0 = s0_16[j]
                    s1 = s1_16[j]
                    for cc in range(DIM // LANES):
                        v0 = bufa_v[_k, r, pl.ds(s0 + cc * LANES, LANES)]
                        v1 = bufb_v[_k, r, pl.ds(s1 + cc * LANES, LANES)]
                        plsc.store_scatter(
                            res_v, [cc * LANES + lane,
                                    jnp.full((LANES,), q, jnp.int32)],
                            v0 * one_m + v1 * wb)
                return c
            lax.fori_loop(0, SUB // LANES, row16, 0)

        pltpu.sync_copy(res_v, out_t_hbm.at[:, pl.ds(off, C2)])
        return carry

    lax.fori_loop(0, C2_CHUNKS, chunk, 0)


@jax.jit
def kernel(input, param):
    mesh = plsc.VectorSubcoreMesh(core_axis_name="c", subcore_axis_name="s")
    params = pltpu.CompilerParams(use_tc_tiling_on_sc=True,
                                  needs_layout_passes=False)

    tab_t = input.T

    pairs = pl.kernel(
        _compact_body,
        out_type=jax.ShapeDtypeStruct((NPAIR, 2 * DIM), jnp.float32),
        mesh=mesh,
        scratch_types=[
            pltpu.VMEM((DIM, C1), jnp.float32),
            pltpu.VMEM((C1 // 2, 2 * DIM), jnp.float32),
        ],
        compiler_params=params,
    )(tab_t)

    pairs = pl.pallas_call(
        _tail_body,
        grid=(1,),
        in_specs=[
            pl.BlockSpec((DIM, 2 * DIM), lambda i: (0, TAIL0 // (2 * DIM))),
            pl.BlockSpec(memory_space=pl.ANY),
        ],
        out_specs=pl.BlockSpec((DIM, 2 * DIM), lambda i: (TAIL0 // (2 * DIM), 0)),
        out_shape=jax.ShapeDtypeStruct((NPAIR, 2 * DIM), jnp.float32),
        input_output_aliases={1: 0},
    )(tab_t, pairs)

    out_t = pl.kernel(
        _sample_body,
        out_type=jax.ShapeDtypeStruct((DIM, BATCH), jnp.float32),
        mesh=mesh,
        scratch_types=[
            pltpu.VMEM((C2,), jnp.float32),
            pltpu.VMEM((C2,), jnp.float32),
            pltpu.VMEM((C2,), jnp.int32),
            pltpu.VMEM((C2,), jnp.int32),
            pltpu.VMEM((KSUB, SUB), jnp.int32),
            pltpu.VMEM((KSUB, SUB), jnp.int32),
            pltpu.VMEM((KSUB, SUB, 2 * DIM), jnp.float32),
            pltpu.VMEM((KSUB, SUB, 2 * DIM), jnp.float32),
            pltpu.VMEM((DIM, C2), jnp.float32),
            pltpu.SemaphoreType.DMA,
        ],
        compiler_params=params,
    )(pairs, param)
    return out_t.T


# speedup vs baseline: 2.6758x; 2.6758x over previous
import jax
import jax.numpy as jnp
from jax import lax
from jax.experimental import pallas as pl
from jax.experimental.pallas import tpu as pltpu
from jax.experimental.pallas import tpu_sc as plsc

N_ROWS = 1_000_000
DIM = 64
BATCH = 819_200

NUM_CORES = 2
NUM_SUBCORES = 16
LANES = 16
NUM_WORKERS = NUM_CORES * NUM_SUBCORES

B_PER_W = BATCH // NUM_WORKERS
CHUNK = 256
SUB = 128
KSUB = CHUNK // SUB
NUM_CHUNKS = B_PER_W // CHUNK


def _sampler_body(table_hbm, param_hbm, out_hbm,
                  param_v0, param_v1, w_v0, w_v1,
                  idx0_v0, idx0_v1, idx1_v0, idx1_v1,
                  rows0_v0, rows0_v1, rows1_v0, rows1_v1,
                  sem_g0, sem_g1, sem_p0, sem_p1, sem_o0, sem_o1):
    wid = lax.axis_index("s") * NUM_CORES + lax.axis_index("c")
    base = wid * B_PER_W
    scale = jnp.float32(N_ROWS - 1)

    param_v = [param_v0, param_v1]
    w_v = [w_v0, w_v1]
    idx0_v = [idx0_v0, idx0_v1]
    idx1_v = [idx1_v0, idx1_v1]
    rows0_v = [rows0_v0, rows0_v1]
    rows1_v = [rows1_v0, rows1_v1]
    sem_g = [sem_g0, sem_g1]
    sem_p = [sem_p0, sem_p1]
    sem_o = [sem_o0, sem_o1]

    def compute_idx(s):
        for j in range(CHUNK // LANES):
            p = param_v[s][pl.ds(j * LANES, LANES)]
            t = jnp.minimum(jnp.maximum(p, 0.0), 1.0) * scale
            i0 = t.astype(jnp.int32)
            i1 = jnp.minimum(i0 + 1, N_ROWS - 1)
            w = t - i0.astype(jnp.float32)
            k, r = divmod(j * LANES, SUB)
            idx0_v[s][k, pl.ds(r, LANES)] = i0
            idx1_v[s][k, pl.ds(r, LANES)] = i1
            w_v[s][pl.ds(j * LANES, LANES)] = w

    def issue_gathers(s):
        for k in range(KSUB):
            pltpu.async_copy(table_hbm.at[idx0_v[s].at[k]],
                             rows0_v[s].at[k], sem_g[s])
            pltpu.async_copy(table_hbm.at[idx1_v[s].at[k]],
                             rows1_v[s].at[k], sem_g[s])

    def wait_gathers(s):
        for k in range(KSUB):
            pltpu.make_async_copy(table_hbm.at[pl.ds(0, SUB), :],
                                  rows0_v[s].at[k], sem_g[s]).wait()
            pltpu.make_async_copy(table_hbm.at[pl.ds(0, SUB), :],
                                  rows1_v[s].at[k], sem_g[s]).wait()

    def issue_param(g, s):
        off = pl.multiple_of(base + g * CHUNK, CHUNK)
        pltpu.async_copy(param_hbm.at[pl.ds(off, CHUNK)], param_v[s],
                         sem_p[s])

    def wait_param(s):
        pltpu.make_async_copy(param_hbm.at[pl.ds(0, CHUNK)], param_v[s],
                              sem_p[s]).wait()

    def lerp(s):
        for k in range(KSUB):
            def row16(r16, c, _k=k):
                w16 = w_v[s][pl.ds(_k * SUB + r16 * LANES, LANES)]
                for j in range(LANES):
                    wb = w16.at[jnp.full((LANES,), j, jnp.int32)].get(
                        mode="promise_in_bounds")
                    one_m = 1.0 - wb
                    r = r16 * LANES + j
                    for cc in range(DIM // LANES):
                        v0 = rows0_v[s][_k, r, pl.ds(cc * LANES, LANES)]
                        v1 = rows1_v[s][_k, r, pl.ds(cc * LANES, LANES)]
                        rows0_v[s][_k, r, pl.ds(cc * LANES, LANES)] = (
                            v0 * one_m + v1 * wb)
                return c
            lax.fori_loop(0, SUB // LANES, row16, 0)

    def issue_out(g, s):
        off = pl.multiple_of(base + g * CHUNK, CHUNK)
        for k in range(KSUB):
            pltpu.async_copy(rows0_v[s].at[k],
                             out_hbm.at[pl.ds(off + k * SUB, SUB)], sem_o[s])

    def wait_out(s):
        for k in range(KSUB):
            pltpu.make_async_copy(rows0_v[s].at[k],
                                  out_hbm.at[pl.ds(0, SUB)], sem_o[s]).wait()

    issue_param(0, 0)
    wait_param(0)
    compute_idx(0)
    issue_gathers(0)
    issue_param(1, 1)

    def half(g, cur, nxt):
        @pl.when(g + 1 < NUM_CHUNKS)
        def _():
            wait_param(nxt)
            compute_idx(nxt)

            @pl.when(g + 1 >= 2)
            def _():
                wait_out(nxt)
            issue_gathers(nxt)

        @pl.when(g + 2 < NUM_CHUNKS)
        def _():
            issue_param(g + 2, cur)

        wait_gathers(cur)
        lerp(cur)
        issue_out(g, cur)

    def body(i, carry):
        half(2 * i, 0, 1)
        half(2 * i + 1, 1, 0)
        return carry

    lax.fori_loop(0, NUM_CHUNKS // 2, body, 0)
    wait_out(0)
    wait_out(1)


@jax.jit
def kernel(input, param):
    mesh = plsc.VectorSubcoreMesh(core_axis_name="c", subcore_axis_name="s")
    f = pl.kernel(
        _sampler_body,
        out_type=jax.ShapeDtypeStruct((BATCH, DIM), jnp.float32),
        mesh=mesh,
        scratch_types=[
            pltpu.VMEM((CHUNK,), jnp.float32),
            pltpu.VMEM((CHUNK,), jnp.float32),
            pltpu.VMEM((CHUNK,), jnp.float32),
            pltpu.VMEM((CHUNK,), jnp.float32),
            pltpu.VMEM((KSUB, SUB), jnp.int32),
            pltpu.VMEM((KSUB, SUB), jnp.int32),
            pltpu.VMEM((KSUB, SUB), jnp.int32),
            pltpu.VMEM((KSUB, SUB), jnp.int32),
            pltpu.VMEM((KSUB, SUB, DIM), jnp.float32),
            pltpu.VMEM((KSUB, SUB, DIM), jnp.float32),
            pltpu.VMEM((KSUB, SUB, DIM), jnp.float32),
            pltpu.VMEM((KSUB, SUB, DIM), jnp.float32),
            pltpu.SemaphoreType.DMA,
            pltpu.SemaphoreType.DMA,
            pltpu.SemaphoreType.DMA,
            pltpu.SemaphoreType.DMA,
            pltpu.SemaphoreType.DMA,
            pltpu.SemaphoreType.DMA,
        ],
        compiler_params=pltpu.CompilerParams(use_tc_tiling_on_sc=False),
    )(input, param)
    return f
